# baseline (speedup 1.0000x reference)
import functools

import jax
import jax.numpy as jnp
from jax import lax
from jax.experimental import pallas as pl
from jax.experimental.pallas import tpu as pltpu
from jax.experimental.pallas import tpu_sc as plsc

_NC = 2
_NS = 16
_NW = _NC * _NS
_CH = 128


def _gather_body(ut_hbm, it_hbm, uidx_hbm, iidx_hbm, ue_hbm, ie_hbm,
                 idx_u, idx_i, rows_u, rows_i, sem, *, bpw, nchunk):
    wid = lax.axis_index("s") * _NC + lax.axis_index("c")
    base = wid * bpw
    pltpu.sync_copy(uidx_hbm.at[wid], idx_u)
    pltpu.sync_copy(iidx_hbm.at[wid], idx_i)
    copies = []
    for j in range(nchunk):
        copies.append(
            pltpu.async_copy(ut_hbm.at[idx_u.at[j]],
                             rows_u.at[pl.ds(j * _CH, _CH)], sem))
        copies.append(
            pltpu.async_copy(it_hbm.at[idx_i.at[j]],
                             rows_i.at[pl.ds(j * _CH, _CH)], sem))
    for c in copies:
        c.wait()
    pltpu.sync_copy(rows_u, ue_hbm.at[pl.ds(base, bpw)])
    pltpu.sync_copy(rows_i, ie_hbm.at[pl.ds(base, bpw)])


def _sc_gather(user_table, item_table, uidx, iidx):
    batch = uidx.shape[0]
    dim = user_table.shape[1]
    bpw = batch // _NW
    nchunk = bpw // _CH
    mesh = plsc.VectorSubcoreMesh(core_axis_name="c", subcore_axis_name="s")
    uidx3 = uidx.astype(jnp.int32).reshape(_NW, nchunk, _CH)
    iidx3 = iidx.astype(jnp.int32).reshape(_NW, nchunk, _CH)
    body = functools.partial(_gather_body, bpw=bpw, nchunk=nchunk)
    fn = pl.kernel(
        body,
        out_type=(jax.ShapeDtypeStruct((batch, dim), jnp.float32),
                  jax.ShapeDtypeStruct((batch, dim), jnp.float32)),
        mesh=mesh,
        scratch_types=[
            pltpu.VMEM((nchunk, _CH), jnp.int32),
            pltpu.VMEM((nchunk, _CH), jnp.int32),
            pltpu.VMEM((bpw, dim), jnp.float32),
            pltpu.VMEM((bpw, dim), jnp.float32),
            pltpu.SemaphoreType.DMA,
        ],
        compiler_params=pltpu.CompilerParams(use_tc_tiling_on_sc=False),
    )
    return fn(user_table, item_table, uidx3, iidx3)


def _mlp_body(ue_ref, ie_ref, w0u_ref, w0i_ref, b0_ref, w1_ref, b1_ref,
              w2_ref, b2_ref, wo_ref, bo_ref, out_ref):
    h = (jnp.dot(ue_ref[...], w0u_ref[...], preferred_element_type=jnp.float32)
         + jnp.dot(ie_ref[...], w0i_ref[...], preferred_element_type=jnp.float32)
         + b0_ref[...])
    h = jnp.maximum(h, 0.0)
    h = jnp.dot(h, w1_ref[...], preferred_element_type=jnp.float32) + b1_ref[...]
    h = jnp.maximum(h, 0.0)
    h = jnp.dot(h, w2_ref[...], preferred_element_type=jnp.float32) + b2_ref[...]
    h = jnp.maximum(h, 0.0)
    logits = jnp.sum(h * wo_ref[...], axis=1, keepdims=True) + bo_ref[...]
    out_ref[...] = jax.nn.sigmoid(logits)


def _mlp(ue, ie, W0, b0, W1, b1, W2, b2, Wo, bo):
    batch, dim = ue.shape
    bt = 2048
    d0 = W0.shape[1]
    d1 = W1.shape[1]
    d2 = W2.shape[1]
    w0u = W0[:dim]
    w0i = W0[dim:]
    out = pl.pallas_call(
        _mlp_body,
        grid=(batch // bt,),
        in_specs=[
            pl.BlockSpec((bt, dim), lambda i: (i, 0)),
            pl.BlockSpec((bt, dim), lambda i: (i, 0)),
            pl.BlockSpec((dim, d0), lambda i: (0, 0)),
            pl.BlockSpec((dim, d0), lambda i: (0, 0)),
            pl.BlockSpec((1, d0), lambda i: (0, 0)),
            pl.BlockSpec((d0, d1), lambda i: (0, 0)),
            pl.BlockSpec((1, d1), lambda i: (0, 0)),
            pl.BlockSpec((d1, d2), lambda i: (0, 0)),
            pl.BlockSpec((1, d2), lambda i: (0, 0)),
            pl.BlockSpec((1, d2), lambda i: (0, 0)),
            pl.BlockSpec((1, 1), lambda i: (0, 0)),
        ],
        out_specs=pl.BlockSpec((bt, 1), lambda i: (i, 0)),
        out_shape=jax.ShapeDtypeStruct((batch, 1), jnp.float32),
    )(ue, ie, w0u, w0i, b0.reshape(1, d0), W1, b1.reshape(1, d1),
      W2, b2.reshape(1, d2), Wo.reshape(1, d2), bo.reshape(1, 1))
    return out[:, 0]


def kernel(user_indices, item_indices, user_table, item_table,
           W0, b0, W1, b1, W2, b2, Wo, bo):
    ue, ie = _sc_gather(user_table, item_table, user_indices, item_indices)
    return _mlp(ue, ie, W0, b0, W1, b1, W2, b2, Wo, bo)


# speedup vs baseline: 1.5772x; 1.5772x over previous
import functools

import jax
import jax.numpy as jnp
from jax import lax
from jax.experimental import pallas as pl
from jax.experimental.pallas import tpu as pltpu
from jax.experimental.pallas import tpu_sc as plsc

_NC = 2
_NS = 16
_NW = _NC * _NS
_CH = 128


def _gather_body(ut_hbm, it_hbm, uidx_hbm, iidx_hbm, ue_hbm, ie_hbm,
                 idx_v, rows, sem, *, bpw):
    wid = lax.axis_index("s") * _NC + lax.axis_index("c")
    base = wid * bpw
    for tab, idxh, outh in ((ut_hbm, uidx_hbm, ue_hbm),
                            (it_hbm, iidx_hbm, ie_hbm)):
        pltpu.sync_copy(idxh.at[wid], idx_v)

        def issue(kb, carry, tab=tab):
            vu = idx_v[pl.ds(kb * 16, 16)]
            for l in range(16):
                pltpu.async_copy(tab.at[vu[l]], rows.at[kb * 16 + l], sem)
            return carry

        lax.fori_loop(0, bpw // 16, issue, 0)
        pltpu.make_async_copy(tab.at[pl.ds(0, bpw)], rows, sem).wait()
        pltpu.sync_copy(rows, outh.at[pl.ds(base, bpw)])


def _sc_gather(user_table, item_table, uidx, iidx):
    batch = uidx.shape[0]
    dim = user_table.shape[1]
    bpw = batch // _NW
    mesh = plsc.VectorSubcoreMesh(core_axis_name="c", subcore_axis_name="s")
    uidx2 = uidx.astype(jnp.int32).reshape(_NW, bpw)
    iidx2 = iidx.astype(jnp.int32).reshape(_NW, bpw)
    body = functools.partial(_gather_body, bpw=bpw)
    fn = pl.kernel(
        body,
        out_type=(jax.ShapeDtypeStruct((batch, dim), jnp.float32),
                  jax.ShapeDtypeStruct((batch, dim), jnp.float32)),
        mesh=mesh,
        scratch_types=[
            pltpu.VMEM((bpw,), jnp.int32),
            pltpu.VMEM((bpw, dim), jnp.float32),
            pltpu.SemaphoreType.DMA,
        ],
    )
    return fn(user_table, item_table, uidx2, iidx2)


def _mlp_body(ue_ref, ie_ref, w0u_ref, w0i_ref, b0_ref, w1_ref, b1_ref,
              w2_ref, b2_ref, wo_ref, bo_ref, out_ref):
    h = (jnp.dot(ue_ref[...], w0u_ref[...], preferred_element_type=jnp.float32)
         + jnp.dot(ie_ref[...], w0i_ref[...], preferred_element_type=jnp.float32)
         + b0_ref[...])
    h = jnp.maximum(h, 0.0)
    h = jnp.dot(h, w1_ref[...], preferred_element_type=jnp.float32) + b1_ref[...]
    h = jnp.maximum(h, 0.0)
    h = jnp.dot(h, w2_ref[...], preferred_element_type=jnp.float32) + b2_ref[...]
    h = jnp.maximum(h, 0.0)
    logits = jnp.sum(h * wo_ref[...], axis=1, keepdims=True) + bo_ref[...]
    out_ref[...] = jax.nn.sigmoid(logits)


def _mlp(ue, ie, W0, b0, W1, b1, W2, b2, Wo, bo):
    batch, dim = ue.shape
    bt = 2048
    d0 = W0.shape[1]
    d1 = W1.shape[1]
    d2 = W2.shape[1]
    w0u = W0[:dim]
    w0i = W0[dim:]
    out = pl.pallas_call(
        _mlp_body,
        grid=(batch // bt,),
        in_specs=[
            pl.BlockSpec((bt, dim), lambda i: (i, 0)),
            pl.BlockSpec((bt, dim), lambda i: (i, 0)),
            pl.BlockSpec((dim, d0), lambda i: (0, 0)),
            pl.BlockSpec((dim, d0), lambda i: (0, 0)),
            pl.BlockSpec((1, d0), lambda i: (0, 0)),
            pl.BlockSpec((d0, d1), lambda i: (0, 0)),
            pl.BlockSpec((1, d1), lambda i: (0, 0)),
            pl.BlockSpec((d1, d2), lambda i: (0, 0)),
            pl.BlockSpec((1, d2), lambda i: (0, 0)),
            pl.BlockSpec((1, d2), lambda i: (0, 0)),
            pl.BlockSpec((1, 1), lambda i: (0, 0)),
        ],
        out_specs=pl.BlockSpec((bt, 1), lambda i: (i, 0)),
        out_shape=jax.ShapeDtypeStruct((batch, 1), jnp.float32),
    )(ue, ie, w0u, w0i, b0.reshape(1, d0), W1, b1.reshape(1, d1),
      W2, b2.reshape(1, d2), Wo.reshape(1, d2), bo.reshape(1, 1))
    return out[:, 0]


def kernel(user_indices, item_indices, user_table, item_table,
           W0, b0, W1, b1, W2, b2, Wo, bo):
    ue, ie = _sc_gather(user_table, item_table, user_indices, item_indices)
    return _mlp(ue, ie, W0, b0, W1, b1, W2, b2, Wo, bo)
